# Initial kernel scaffold; baseline (speedup 1.0000x reference)
import jax
import jax.numpy as jnp
from jax.experimental import pallas as pl


def kernel(y_pred, y_true):
    raise NotImplementedError("write your pallas kernel here")


# speedup vs baseline: 1.0072x; 1.0072x over previous
import jax
import jax.numpy as jnp
from jax.experimental import pallas as pl
from jax.experimental.pallas import tpu as pltpu

_MARGIN = 0.5
_N = 4096
_ROWS = 256
_GRID = _N // _ROWS
_BIG = 1e30


def _pair_kernel(pc_ref, tc_ref, pr_ref, tr_ref, out_ref, acc_ref):
    i = pl.program_id(0)

    @pl.when(i == 0)
    def _init():
        acc_ref[0, 0] = jnp.float32(0.0)

    pcol = pc_ref[...]
    tcol = tc_ref[...]
    prow = pr_ref[...]
    trow = tr_ref[...]

    pos_vals = jnp.where(tcol == 1, pcol, jnp.float32(_BIG))
    neg_vals = jnp.where(trow == 0, prow + jnp.float32(_MARGIN), jnp.float32(-_BIG))
    d = neg_vals - pos_vals
    acc_ref[0, 0] += jnp.sum(jnp.maximum(d, jnp.float32(0.0)))

    @pl.when(i == _GRID - 1)
    def _finish():
        npos = jnp.sum((trow == 1).astype(jnp.float32))
        nneg = jnp.sum((trow == 0).astype(jnp.float32))
        denom = npos * nneg
        total = acc_ref[0, 0]
        out_ref[0, 0] = jnp.where(
            denom > 0, total / jnp.maximum(denom, jnp.float32(1.0)), jnp.float32(0.0)
        )


def kernel(y_pred, y_true):
    pc = y_pred.reshape(_N, 1)
    tc = y_true.reshape(_N, 1)
    pr = y_pred.reshape(1, _N)
    tr = y_true.reshape(1, _N)
    out = pl.pallas_call(
        _pair_kernel,
        grid=(_GRID,),
        in_specs=[
            pl.BlockSpec((_ROWS, 1), lambda i: (i, 0)),
            pl.BlockSpec((_ROWS, 1), lambda i: (i, 0)),
            pl.BlockSpec((1, _N), lambda i: (0, 0)),
            pl.BlockSpec((1, _N), lambda i: (0, 0)),
        ],
        out_specs=pl.BlockSpec(memory_space=pltpu.SMEM),
        out_shape=jax.ShapeDtypeStruct((1, 1), jnp.float32),
        scratch_shapes=[pltpu.SMEM((1, 1), jnp.float32)],
    )(pc, tc, pr, tr)
    return out[0, 0]
